# baseline (speedup 1.0000x reference)
import functools

import jax
import jax.numpy as jnp
from jax import lax
from jax.experimental import pallas as pl
from jax.experimental.pallas import tpu as pltpu
from jax.experimental.pallas import tpu_sc as plsc

VOCAB = 1000000
SEQ_LEN = 200
EMBED_DIM = 32
BATCH = 4096

NUM_CORES = 2
NUM_SUBCORES = 16
NUM_WORKERS = NUM_CORES * NUM_SUBCORES

LANES = 16
BB = 128
NBLK = BATCH // BB
DB = 8
NG = EMBED_DIM // DB
GRP = 8
NGRP = NBLK // GRP

L_BASE = SEQ_LEN // NUM_WORKERS
L_EXTRA = SEQ_LEN % NUM_WORKERS


def _body(idx_hbm, tok_hbm, pos_hbm, out_hbm,
          idx_v, r0_v, r1_v, o_v, pos_v, psplat_v, sem0, sem1):
    wid = lax.axis_index("s") * NUM_CORES + lax.axis_index("c")
    pltpu.sync_copy(pos_hbm, pos_v)

    lo = wid * L_BASE + jnp.minimum(wid, L_EXTRA)
    cnt = L_BASE + jnp.where(wid < L_EXTRA, 1, 0)

    iota = lax.iota(jnp.int32, LANES)
    zeros16 = jnp.zeros((LANES,), jnp.int32)

    r_bufs = (r0_v, r1_v)
    sems = (sem0, sem1)

    def fire_group(grp, rbuf, sem):
        handles = []
        for j in range(GRP):
            c = grp * GRP + j
            handles.append(
                pltpu.async_copy(tok_hbm.at[idx_v.at[c]], rbuf.at[j], sem)
            )
        return handles

    def l_body(l, _):
        pltpu.sync_copy(idx_hbm.at[l], idx_v)
        l_vec = zeros16 + l

        for d in range(EMBED_DIM):
            d_vec = jnp.full((LANES,), d, jnp.int32)
            psplat_v[d, :] = plsc.load_gather(pos_v, [l_vec, d_vec])

        handles = fire_group(0, r_bufs[0], sems[0])
        for grp in range(NGRP):
            rbuf = r_bufs[grp % 2]
            next_handles = None
            if grp + 1 < NGRP:
                next_handles = fire_group(
                    grp + 1, r_bufs[(grp + 1) % 2], sems[(grp + 1) % 2]
                )
            for h in handles:
                h.wait()
            handles = next_handles

            def cc_body(cc, carry):
                rblk = rbuf.at[cc]

                def d_body(d, carry2):
                    d_vec = zeros16 + d
                    g = d // DB
                    dd = lax.rem(d, DB)
                    pvec = psplat_v[d, :]
                    for hh in range(BB // LANES):
                        row_idx = iota + (hh * LANES)
                        vals = plsc.load_gather(rblk, [row_idx, d_vec])
                        o_v[g, cc, dd, pl.ds(hh * LANES, LANES)] = vals + pvec
                    return carry2

                lax.fori_loop(0, EMBED_DIM, d_body, 0)
                return carry

            lax.fori_loop(0, GRP, cc_body, 0)

            for g in range(NG):
                pltpu.sync_copy(
                    o_v.at[g], out_hbm.at[l, g, pl.ds(grp * GRP, GRP)]
                )
        return _

    lax.fori_loop(lo, lo + cnt, l_body, 0)


_mesh = plsc.VectorSubcoreMesh(core_axis_name="c", subcore_axis_name="s")

_sc_call = functools.partial(
    pl.kernel,
    out_type=jax.ShapeDtypeStruct((SEQ_LEN, NG, NBLK, DB, BB), jnp.float32),
    mesh=_mesh,
    scratch_types=[
        pltpu.VMEM((NBLK, BB), jnp.int32),
        pltpu.VMEM((GRP, BB, EMBED_DIM), jnp.float32),
        pltpu.VMEM((GRP, BB, EMBED_DIM), jnp.float32),
        pltpu.VMEM((NG, GRP, DB, BB), jnp.float32),
        pltpu.VMEM((SEQ_LEN, EMBED_DIM), jnp.float32),
        pltpu.VMEM((EMBED_DIM, LANES), jnp.float32),
        pltpu.SemaphoreType.DMA,
        pltpu.SemaphoreType.DMA,
    ],
    compiler_params=pltpu.CompilerParams(
        use_tc_tiling_on_sc=False, needs_layout_passes=False
    ),
)


@jax.jit
def kernel(inputs, token_table, pos_table):
    idx = inputs.astype(jnp.int32).T.reshape(SEQ_LEN, NBLK, BB)
    o5 = _sc_call(_body)(idx, token_table, pos_table)
    out = o5.transpose(0, 1, 3, 2, 4).reshape(SEQ_LEN, EMBED_DIM, BATCH)
    return out.transpose(2, 0, 1)
